# baseline (speedup 1.0000x reference)
import functools
import math

import jax
import jax.numpy as jnp
from jax import lax
from jax.experimental import pallas as pl
from jax.experimental.pallas import tpu as pltpu
from jax.experimental.pallas import tpu_sc as plsc

VOCAB_ROWS = 100000
EMB_DIM = 128
SCALE = math.sqrt(float(EMB_DIM))

NUM_CORES = 2
NUM_SUBCORES = 16
NW = NUM_CORES * NUM_SUBCORES

CHUNK = 128


def _scale_body(t_ref, o_ref):
    o_ref[...] = t_ref[...] * SCALE


def _scale_table(table):
    rows = table.shape[0]
    block = 1000
    return pl.pallas_call(
        _scale_body,
        grid=(rows // block,),
        in_specs=[pl.BlockSpec((block, EMB_DIM), lambda i: (i, 0))],
        out_specs=pl.BlockSpec((block, EMB_DIM), lambda i: (i, 0)),
        out_shape=jax.ShapeDtypeStruct((rows, EMB_DIM), jnp.float32),
    )(table)


def _make_gather(n_tokens):
    assert n_tokens % (NW * CHUNK) == 0
    bpw = n_tokens // NW
    n_chunks = bpw // CHUNK

    mesh = plsc.VectorSubcoreMesh(core_axis_name="c", subcore_axis_name="s")

    NBUF = 4
    n_quads = n_chunks // NBUF
    assert n_chunks % NBUF == 0 and n_quads >= 2

    @functools.partial(
        pl.kernel,
        mesh=mesh,
        out_type=jax.ShapeDtypeStruct((n_tokens, EMB_DIM), jnp.float32),
        scratch_types=[
            pltpu.VMEM((n_chunks, CHUNK), jnp.int32),
        ]
        + [pltpu.VMEM((CHUNK, EMB_DIM), jnp.float32) for _ in range(NBUF)]
        + [pltpu.SemaphoreType.DMA for _ in range(2 * NBUF)],
    )
    def gather_kernel(idx_hbm, table_hbm, out_hbm, idx_v, *rest):
        bufs = rest[:NBUF]
        sg = rest[NBUF : 2 * NBUF]
        so = rest[2 * NBUF : 3 * NBUF]
        wid = lax.axis_index("s") * NUM_CORES + lax.axis_index("c")
        base = wid * bpw
        pltpu.sync_copy(idx_hbm.at[wid], idx_v)

        def start_gather(g, j):
            pltpu.async_copy(table_hbm.at[idx_v.at[g]], bufs[j], sg[j])

        def wait_gather(g, j):
            pltpu.make_async_copy(table_hbm.at[idx_v.at[g]], bufs[j], sg[j]).wait()

        def start_out(g, j):
            pltpu.async_copy(bufs[j], out_hbm.at[pl.ds(base + g * CHUNK, CHUNK)], so[j])

        def wait_out(g, j):
            pltpu.make_async_copy(
                bufs[j], out_hbm.at[pl.ds(base + g * CHUNK, CHUNK)], so[j]
            ).wait()

        start_gather(0, 0)
        start_gather(1, 1)
        start_gather(2, 2)
        wait_gather(0, 0)
        start_out(0, 0)
        start_gather(3, 3)
        wait_gather(1, 1)
        start_out(1, 1)

        def body(q, carry):
            g0 = q * NBUF
            for j in range(NBUF):
                g = g0 + j
                wait_out(g - NBUF, j)
                start_gather(g, j)
                j2 = (j + 2) % NBUF
                wait_gather(g - 2, j2)
                start_out(g - 2, j2)
            return carry

        lax.fori_loop(1, n_quads, body, 0)

        last = n_chunks - NBUF
        wait_gather(n_chunks - 2, 2)
        start_out(n_chunks - 2, 2)
        wait_gather(n_chunks - 1, 3)
        start_out(n_chunks - 1, 3)
        for j in range(NBUF):
            wait_out(last + j, j)

    return gather_kernel


def kernel(tokens, table):
    n_tokens = tokens.shape[0] * tokens.shape[1]
    idx = tokens.reshape(NW, n_tokens // (NW * CHUNK), CHUNK).astype(jnp.int32)
    scaled = _scale_table(table)
    out = _make_gather(n_tokens)(idx, scaled)
    return out.reshape(tokens.shape[0], tokens.shape[1], EMB_DIM)


# speedup vs baseline: 1.1011x; 1.1011x over previous
import functools
import math

import jax
import jax.numpy as jnp
from jax import lax
from jax.experimental import pallas as pl
from jax.experimental.pallas import tpu as pltpu
from jax.experimental.pallas import tpu_sc as plsc

VOCAB_ROWS = 100000
EMB_DIM = 128
SCALE = math.sqrt(float(EMB_DIM))

NUM_CORES = 2
NUM_SUBCORES = 16
NW = NUM_CORES * NUM_SUBCORES

CHUNK = 128


def _scale_body(t_ref, o_ref):
    o_ref[...] = t_ref[...] * SCALE


def _scale_table(table):
    rows = table.shape[0]
    block = 4000
    return pl.pallas_call(
        _scale_body,
        grid=(rows // block,),
        in_specs=[pl.BlockSpec((block, EMB_DIM), lambda i: (i, 0))],
        out_specs=pl.BlockSpec((block, EMB_DIM), lambda i: (i, 0)),
        out_shape=jax.ShapeDtypeStruct((rows, EMB_DIM), jnp.float32),
    )(table)


def _make_gather(n_tokens):
    assert n_tokens % (NW * CHUNK) == 0
    bpw = n_tokens // NW
    n_chunks = bpw // CHUNK

    mesh = plsc.VectorSubcoreMesh(core_axis_name="c", subcore_axis_name="s")

    NBUF = 4
    n_quads = n_chunks // NBUF
    assert n_chunks % NBUF == 0 and n_quads >= 2

    @functools.partial(
        pl.kernel,
        mesh=mesh,
        out_type=jax.ShapeDtypeStruct((n_tokens, EMB_DIM), jnp.float32),
        scratch_types=[
            pltpu.VMEM((n_chunks, CHUNK), jnp.int32),
        ]
        + [pltpu.VMEM((CHUNK, EMB_DIM), jnp.float32) for _ in range(NBUF)]
        + [pltpu.SemaphoreType.DMA for _ in range(2 * NBUF)],
    )
    def gather_kernel(idx_hbm, table_hbm, out_hbm, idx_v, *rest):
        bufs = rest[:NBUF]
        sg = rest[NBUF : 2 * NBUF]
        so = rest[2 * NBUF : 3 * NBUF]
        wid = lax.axis_index("s") * NUM_CORES + lax.axis_index("c")
        base = wid * bpw
        pltpu.sync_copy(idx_hbm.at[wid], idx_v)

        def start_gather(g, j):
            pltpu.async_copy(table_hbm.at[idx_v.at[g]], bufs[j], sg[j])

        def wait_gather(g, j):
            pltpu.make_async_copy(table_hbm.at[idx_v.at[g]], bufs[j], sg[j]).wait()

        def start_out(g, j):
            pltpu.async_copy(bufs[j], out_hbm.at[pl.ds(base + g * CHUNK, CHUNK)], so[j])

        def wait_out(g, j):
            pltpu.make_async_copy(
                bufs[j], out_hbm.at[pl.ds(base + g * CHUNK, CHUNK)], so[j]
            ).wait()

        start_gather(0, 0)
        start_gather(1, 1)
        start_gather(2, 2)
        wait_gather(0, 0)
        start_out(0, 0)
        start_gather(3, 3)
        wait_gather(1, 1)
        start_out(1, 1)

        def body(q, carry):
            g0 = q * NBUF
            for j in range(NBUF):
                g = g0 + j
                wait_out(g - NBUF, j)
                start_gather(g, j)
                j2 = (j + 2) % NBUF
                wait_gather(g - 2, j2)
                start_out(g - 2, j2)
            return carry

        lax.fori_loop(1, n_quads, body, 0)

        last = n_chunks - NBUF
        wait_gather(n_chunks - 2, 2)
        start_out(n_chunks - 2, 2)
        wait_gather(n_chunks - 1, 3)
        start_out(n_chunks - 1, 3)
        for j in range(NBUF):
            wait_out(last + j, j)

    return gather_kernel


def kernel(tokens, table):
    n_tokens = tokens.shape[0] * tokens.shape[1]
    idx = tokens.reshape(NW, n_tokens // (NW * CHUNK), CHUNK).astype(jnp.int32)
    scaled = _scale_table(table)
    out = _make_gather(n_tokens)(idx, scaled)
    return out.reshape(tokens.shape[0], tokens.shape[1], EMB_DIM)


# speedup vs baseline: 1.1139x; 1.0117x over previous
import functools
import math

import jax
import jax.numpy as jnp
from jax import lax
from jax.experimental import pallas as pl
from jax.experimental.pallas import tpu as pltpu
from jax.experimental.pallas import tpu_sc as plsc

VOCAB_ROWS = 100000
EMB_DIM = 128
SCALE = math.sqrt(float(EMB_DIM))

NUM_CORES = 2
NUM_SUBCORES = 16
NW = NUM_CORES * NUM_SUBCORES

CHUNK = 128


def _scale_body(t_ref, o_ref):
    o_ref[...] = t_ref[...] * SCALE


def _scale_table(table):
    rows = table.shape[0]
    block = 10000
    return pl.pallas_call(
        _scale_body,
        grid=(rows // block,),
        in_specs=[pl.BlockSpec((block, EMB_DIM), lambda i: (i, 0))],
        out_specs=pl.BlockSpec((block, EMB_DIM), lambda i: (i, 0)),
        out_shape=jax.ShapeDtypeStruct((rows, EMB_DIM), jnp.float32),
    )(table)


def _make_gather(n_tokens):
    assert n_tokens % (NW * CHUNK) == 0
    bpw = n_tokens // NW
    n_chunks = bpw // CHUNK

    mesh = plsc.VectorSubcoreMesh(core_axis_name="c", subcore_axis_name="s")

    NBUF = 4
    n_quads = n_chunks // NBUF
    assert n_chunks % NBUF == 0 and n_quads >= 2

    @functools.partial(
        pl.kernel,
        mesh=mesh,
        out_type=jax.ShapeDtypeStruct((n_tokens, EMB_DIM), jnp.float32),
        scratch_types=[
            pltpu.VMEM((n_chunks, CHUNK), jnp.int32),
        ]
        + [pltpu.VMEM((CHUNK, EMB_DIM), jnp.float32) for _ in range(NBUF)]
        + [pltpu.SemaphoreType.DMA for _ in range(2 * NBUF)],
    )
    def gather_kernel(idx_hbm, table_hbm, out_hbm, idx_v, *rest):
        bufs = rest[:NBUF]
        sg = rest[NBUF : 2 * NBUF]
        so = rest[2 * NBUF : 3 * NBUF]
        wid = lax.axis_index("s") * NUM_CORES + lax.axis_index("c")
        base = wid * bpw
        pltpu.sync_copy(idx_hbm.at[wid], idx_v)

        def start_gather(g, j):
            pltpu.async_copy(table_hbm.at[idx_v.at[g]], bufs[j], sg[j])

        def wait_gather(g, j):
            pltpu.make_async_copy(table_hbm.at[idx_v.at[g]], bufs[j], sg[j]).wait()

        def start_out(g, j):
            pltpu.async_copy(bufs[j], out_hbm.at[pl.ds(base + g * CHUNK, CHUNK)], so[j])

        def wait_out(g, j):
            pltpu.make_async_copy(
                bufs[j], out_hbm.at[pl.ds(base + g * CHUNK, CHUNK)], so[j]
            ).wait()

        start_gather(0, 0)
        start_gather(1, 1)
        start_gather(2, 2)
        wait_gather(0, 0)
        start_out(0, 0)
        start_gather(3, 3)
        wait_gather(1, 1)
        start_out(1, 1)

        def body(q, carry):
            g0 = q * NBUF
            for j in range(NBUF):
                g = g0 + j
                wait_out(g - NBUF, j)
                start_gather(g, j)
                j2 = (j + 2) % NBUF
                wait_gather(g - 2, j2)
                start_out(g - 2, j2)
            return carry

        lax.fori_loop(1, n_quads, body, 0)

        last = n_chunks - NBUF
        wait_gather(n_chunks - 2, 2)
        start_out(n_chunks - 2, 2)
        wait_gather(n_chunks - 1, 3)
        start_out(n_chunks - 1, 3)
        for j in range(NBUF):
            wait_out(last + j, j)

    return gather_kernel


def kernel(tokens, table):
    n_tokens = tokens.shape[0] * tokens.shape[1]
    idx = tokens.reshape(NW, n_tokens // (NW * CHUNK), CHUNK).astype(jnp.int32)
    scaled = _scale_table(table)
    out = _make_gather(n_tokens)(idx, scaled)
    return out.reshape(tokens.shape[0], tokens.shape[1], EMB_DIM)


# speedup vs baseline: 1.2186x; 1.0939x over previous
import functools
import math

import jax
import jax.numpy as jnp
from jax import lax
from jax.experimental import pallas as pl
from jax.experimental.pallas import tpu as pltpu
from jax.experimental.pallas import tpu_sc as plsc

VOCAB_ROWS = 100000
EMB_DIM = 128
SCALE = math.sqrt(float(EMB_DIM))

NUM_CORES = 2
NUM_SUBCORES = 16
NW = NUM_CORES * NUM_SUBCORES

CHUNK = 128


def _make_gather(n_tokens):
    assert n_tokens % (NW * CHUNK) == 0
    bpw = n_tokens // NW
    n_chunks = bpw // CHUNK

    mesh = plsc.VectorSubcoreMesh(core_axis_name="c", subcore_axis_name="s")

    NBUF = 4
    n_quads = n_chunks // NBUF
    assert n_chunks % NBUF == 0 and n_quads >= 2

    @functools.partial(
        pl.kernel,
        mesh=mesh,
        out_type=jax.ShapeDtypeStruct((n_tokens, EMB_DIM), jnp.float32),
        scratch_types=[
            pltpu.VMEM((n_chunks, CHUNK), jnp.int32),
        ]
        + [pltpu.VMEM((CHUNK, EMB_DIM), jnp.float32) for _ in range(NBUF)]
        + [pltpu.SemaphoreType.DMA for _ in range(2 * NBUF)],
    )
    def gather_kernel(idx_hbm, table_hbm, out_hbm, idx_v, *rest):
        bufs = rest[:NBUF]
        sg = rest[NBUF : 2 * NBUF]
        so = rest[2 * NBUF : 3 * NBUF]
        wid = lax.axis_index("s") * NUM_CORES + lax.axis_index("c")
        base = wid * bpw
        pltpu.sync_copy(idx_hbm.at[wid], idx_v)

        def start_gather(g, j):
            pltpu.async_copy(table_hbm.at[idx_v.at[g]], bufs[j], sg[j])

        def wait_gather(g, j):
            pltpu.make_async_copy(table_hbm.at[idx_v.at[g]], bufs[j], sg[j]).wait()

        def scale_buf(j):
            buf = bufs[j]

            def rbody(r, carry):
                for c in range(EMB_DIM // 16):
                    sl = pl.ds(c * 16, 16)
                    buf[r, sl] = buf[r, sl] * SCALE
                return carry

            lax.fori_loop(0, CHUNK, rbody, 0)

        def start_out(g, j):
            pltpu.async_copy(bufs[j], out_hbm.at[pl.ds(base + g * CHUNK, CHUNK)], so[j])

        def wait_out(g, j):
            pltpu.make_async_copy(
                bufs[j], out_hbm.at[pl.ds(base + g * CHUNK, CHUNK)], so[j]
            ).wait()

        start_gather(0, 0)
        start_gather(1, 1)
        start_gather(2, 2)
        wait_gather(0, 0)
        scale_buf(0)
        start_out(0, 0)
        start_gather(3, 3)
        wait_gather(1, 1)
        scale_buf(1)
        start_out(1, 1)

        def body(q, carry):
            g0 = q * NBUF
            for j in range(NBUF):
                g = g0 + j
                wait_out(g - NBUF, j)
                start_gather(g, j)
                j2 = (j + 2) % NBUF
                wait_gather(g - 2, j2)
                scale_buf(j2)
                start_out(g - 2, j2)
            return carry

        lax.fori_loop(1, n_quads, body, 0)

        last = n_chunks - NBUF
        wait_gather(n_chunks - 2, 2)
        scale_buf(2)
        start_out(n_chunks - 2, 2)
        wait_gather(n_chunks - 1, 3)
        scale_buf(3)
        start_out(n_chunks - 1, 3)
        for j in range(NBUF):
            wait_out(last + j, j)

    return gather_kernel


def kernel(tokens, table):
    n_tokens = tokens.shape[0] * tokens.shape[1]
    idx = tokens.reshape(NW, n_tokens // (NW * CHUNK), CHUNK).astype(jnp.int32)
    out = _make_gather(n_tokens)(idx, table)
    return out.reshape(tokens.shape[0], tokens.shape[1], EMB_DIM)
